# Initial kernel scaffold; baseline (speedup 1.0000x reference)
import jax
import jax.numpy as jnp
from jax.experimental import pallas as pl


def kernel(x, edge_index, edge_attr, batch, ne_W1, ne_b1, ne_g1, ne_be1, ne_W2, ne_b2, ne_g2, ne_be2, conv_W, conv_b, conv_mg, conv_mb, conv_ng, conv_nb, outer_g, outer_b, mlp_W1, mlp_b1, mlp_g1, mlp_be1, mlp_W2, mlp_b2, mlp_g2, mlp_be2, mlp_W3, mlp_b3):
    raise NotImplementedError("write your pallas kernel here")


# speedup vs baseline: 3.1373x; 3.1373x over previous
import functools

import jax
import jax.numpy as jnp
from jax import lax
from jax.experimental import pallas as pl
from jax.experimental.pallas import tpu as pltpu
from jax.experimental.pallas import tpu_sc as plsc

_N = 10000
_E = 320000
_D = 128
_L = 3
_G = 16

_NC = 2
_NS = 16
_NW = _NC * _NS
_EPW = _E // _NW
_CG = 80
_NCHUNK = _EPW // _CG
_NP = 10240
_RPT = _NP // _NS

_BN = 2000
_BE = 3200
_BH = 1000

_F32 = jnp.float32
_HIGH = lax.Precision.HIGHEST


def _ln(h, g, b):
    m = jnp.mean(h, axis=-1, keepdims=True)
    v = jnp.mean((h - m) ** 2, axis=-1, keepdims=True)
    return (h - m) * lax.rsqrt(v + 1e-5) * g + b


def _gelu(x):
    return 0.5 * x * (1.0 + lax.erf(x * 0.7071067811865476))


def _enc_body(x_ref, w1_ref, b1_ref, g1_ref, be1_ref, w2_ref, b2_ref, g2_ref,
              be2_ref, wx_ref, bx_ref, h_ref, p_ref):
    t = jnp.dot(x_ref[...], w1_ref[...], precision=_HIGH) + b1_ref[...]
    t = _gelu(_ln(t, g1_ref[...], be1_ref[...]))
    t = jnp.dot(t, w2_ref[...], precision=_HIGH) + b2_ref[...]
    h = _gelu(_ln(t, g2_ref[...], be2_ref[...]))
    h_ref[...] = h
    p_ref[...] = jnp.dot(h, wx_ref[...], precision=_HIGH) + bx_ref[...]


def _edge_body(g_ref, ea_ref, we_ref, mg_ref, mb_ref, m0_ref, m1_ref):
    e2 = jnp.dot(ea_ref[...], we_ref[...], precision=_HIGH)
    m = g_ref[...] + e2
    y = _gelu(_ln(m, mg_ref[...], mb_ref[...]))
    m0_ref[...] = y[:, : _D // 2]
    m1_ref[...] = y[:, _D // 2 :]


def _comb_body(accl_ref, accr_ref, cnt_ref, h_ref, ng_ref, nb_ref, og_ref,
               ob_ref, wx_ref, bx_ref, h2_ref, p_ref):
    s = jnp.concatenate([accl_ref[...], accr_ref[...]], axis=-1)
    cnt = cnt_ref[0, :, 0:1] + cnt_ref[1, :, 0:1]
    agg = s / jnp.maximum(cnt, 1.0)
    h = h_ref[...]
    t = _ln(agg + h, ng_ref[...], nb_ref[...])
    hn = h + _ln(t, og_ref[...], ob_ref[...])
    h2_ref[...] = hn
    p_ref[...] = jnp.dot(hn, wx_ref[...], precision=_HIGH) + bx_ref[...]


def _head_body(h_ref, b_ref, w1_ref, b1_ref, g1_ref, be1_ref, w2_ref, b2_ref,
               g2_ref, be2_ref, w3_ref, b3_ref, out_ref, pool_ref, gc_ref):
    i = pl.program_id(0)
    oh = (lax.broadcasted_iota(jnp.int32, (_G, _BH), 0) == b_ref[0]).astype(_F32)
    part = lax.dot_general(oh, h_ref[...], (((1,), (0,)), ((), ())),
                           precision=_HIGH)
    pc = jnp.sum(oh, axis=1, keepdims=True)

    @pl.when(i == 0)
    def _():
        pool_ref[...] = jnp.zeros_like(pool_ref)
        gc_ref[...] = jnp.zeros_like(gc_ref)

    pool_ref[...] += part
    gc_ref[...] += pc

    @pl.when(i == pl.num_programs(0) - 1)
    def _():
        pooled = pool_ref[...] / jnp.maximum(gc_ref[...], 1.0)
        z = jnp.dot(pooled, w1_ref[...], precision=_HIGH) + b1_ref[...]
        z = jnp.maximum(_ln(z, g1_ref[...], be1_ref[...]), 0.0)
        z = jnp.dot(z, w2_ref[...], precision=_HIGH) + b2_ref[...]
        z = jnp.maximum(_ln(z, g2_ref[...], be2_ref[...]), 0.0)
        out_ref[...] = jnp.dot(z, w3_ref[...], precision=_HIGH) + b3_ref[...]


def _full(shape):
    return pl.BlockSpec(shape, lambda i: (0,) * len(shape))


def _rows(block, width):
    return pl.BlockSpec((block, width), lambda i: (i, 0))


def _enc_call(x, w1, b1, g1, be1, w2, b2, g2, be2, wx, bx):
    full_w = _full((_D, _D))
    full_v = _full((1, _D))
    return pl.pallas_call(
        _enc_body,
        grid=(_N // _BN,),
        in_specs=[_rows(_BN, _D), full_w, full_v, full_v, full_v, full_w,
                  full_v, full_v, full_v, full_w, full_v],
        out_specs=[_rows(_BN, _D), _rows(_BN, _D)],
        out_shape=[jax.ShapeDtypeStruct((_N, _D), _F32),
                   jax.ShapeDtypeStruct((_N, _D), _F32)],
    )(x, w1, b1, g1, be1, w2, b2, g2, be2, wx, bx)


def _edge_call(g, ea, we, mg, mb):
    return pl.pallas_call(
        _edge_body,
        grid=(_E // _BE,),
        in_specs=[_rows(_BE, _D), _rows(_BE, 2), _full((2, _D)),
                  _full((1, _D)), _full((1, _D))],
        out_specs=[_rows(_BE, _D // 2), _rows(_BE, _D // 2)],
        out_shape=[jax.ShapeDtypeStruct((_E, _D // 2), _F32),
                   jax.ShapeDtypeStruct((_E, _D // 2), _F32)],
    )(g, ea, we, mg, mb)


def _comb_call(accl, accr, cnt, h, ng, nb, og, ob, wx, bx):
    full_v = _full((1, _D))
    return pl.pallas_call(
        _comb_body,
        grid=(_N // _BN,),
        in_specs=[_rows(_BN, _D // 2), _rows(_BN, _D // 2),
                  pl.BlockSpec((2, _BN, 16), lambda i: (0, i, 0)),
                  _rows(_BN, _D), full_v, full_v, full_v, full_v,
                  _full((_D, _D)), full_v],
        out_specs=[_rows(_BN, _D), _rows(_BN, _D)],
        out_shape=[jax.ShapeDtypeStruct((_N, _D), _F32),
                   jax.ShapeDtypeStruct((_N, _D), _F32)],
    )(accl, accr, cnt, h, ng, nb, og, ob, wx, bx)


def _head_call(h, batch2, w1, b1, g1, be1, w2, b2, g2, be2, w3, b3):
    return pl.pallas_call(
        _head_body,
        grid=(_N // _BH,),
        in_specs=[_rows(_BH, _D), pl.BlockSpec((1, 1, _BH), lambda i: (i, 0, 0)),
                  _full((_D, _D)), _full((1, _D)), _full((1, _D)),
                  _full((1, _D)), _full((_D, _D // 2)), _full((1, _D // 2)),
                  _full((1, _D // 2)), _full((1, _D // 2)),
                  _full((_D // 2, 1)), _full((1, 1))],
        out_specs=_full((_G, 1)),
        out_shape=jax.ShapeDtypeStruct((_G, 1), _F32),
        scratch_shapes=[pltpu.VMEM((_G, _D), _F32), pltpu.VMEM((_G, 1), _F32)],
    )(h, batch2, w1, b1, g1, be1, w2, b2, g2, be2, w3, b3)


@functools.cache
def _mesh():
    return plsc.VectorSubcoreMesh(core_axis_name="c", subcore_axis_name="s")


def _sc_gather(p, src):
    @functools.partial(
        pl.kernel, mesh=_mesh(),
        out_type=jax.ShapeDtypeStruct((_E, _D), _F32),
        scratch_types=[pltpu.VMEM((_CG,), jnp.int32),
                       pltpu.VMEM((_CG, _D), _F32),
                       pltpu.SemaphoreType.DMA],
    )
    def k(p_hbm, src_hbm, out_hbm, idx_v, rows_v, sem):
        wid = lax.axis_index("s") * _NC + lax.axis_index("c")
        base = wid * _EPW

        def body(i, c):
            off = base + i * _CG
            pltpu.sync_copy(src_hbm.at[pl.ds(off, _CG)], idx_v)
            pltpu.async_copy(p_hbm.at[idx_v], rows_v, sem).wait()
            pltpu.sync_copy(rows_v, out_hbm.at[pl.ds(off, _CG)])
            return c

        lax.fori_loop(0, _NCHUNK, body, 0)

    return k(p, src)


def _sc_scatter(m0, m1, dst, zero_half):
    ept = _E // _NS
    nch = ept // _CG

    @functools.partial(
        pl.kernel, mesh=_mesh(),
        out_type=jax.ShapeDtypeStruct((2 * _NP, _D // 2), _F32),
        scratch_types=[pltpu.VMEM((_CG,), jnp.int32),
                       pltpu.VMEM((_CG, _D // 2), _F32),
                       pltpu.VMEM_SHARED((_NP, _D // 2), _F32)],
    )
    def k(m0_hbm, m1_hbm, dst_hbm, z_hbm, out_hbm, idx_v, upd_v, acc_sh):
        cid = lax.axis_index("c")
        sid = lax.axis_index("s")
        pltpu.sync_copy(z_hbm.at[pl.ds(sid * _RPT, _RPT)],
                        acc_sh.at[pl.ds(sid * _RPT, _RPT)])
        plsc.subcore_barrier()

        def mk_body(m_hbm):
            def body(i, c):
                off = sid * ept + i * _CG
                pltpu.sync_copy(dst_hbm.at[pl.ds(off, _CG)], idx_v)
                pltpu.sync_copy(m_hbm.at[pl.ds(off, _CG)], upd_v)
                pltpu.sync_copy(upd_v, acc_sh.at[idx_v], add=True)
                return c
            return body

        @pl.when(cid == 0)
        def _():
            lax.fori_loop(0, nch, mk_body(m0_hbm), 0)

        @pl.when(cid == 1)
        def _():
            lax.fori_loop(0, nch, mk_body(m1_hbm), 0)

        plsc.subcore_barrier()
        pltpu.sync_copy(acc_sh.at[pl.ds(sid * _RPT, _RPT)],
                        out_hbm.at[pl.ds(cid * _NP + sid * _RPT, _RPT)])

    return k(m0, m1, dst, zero_half)


def _sc_count(dst, pattern, zero_n16):
    @functools.partial(
        pl.kernel, mesh=_mesh(),
        out_type=jax.ShapeDtypeStruct((2 * _NP, 16), _F32),
        scratch_types=[pltpu.VMEM((_CG,), jnp.int32),
                       pltpu.VMEM((_CG, 16), _F32),
                       pltpu.VMEM_SHARED((_NP, 16), _F32)],
    )
    def k(dst_hbm, pat_hbm, z_hbm, out_hbm, idx_v, upd_v, acc_sh):
        cid = lax.axis_index("c")
        sid = lax.axis_index("s")
        wid = sid * _NC + cid
        pltpu.sync_copy(pat_hbm, upd_v)
        pltpu.sync_copy(z_hbm.at[pl.ds(sid * _RPT, _RPT)],
                        acc_sh.at[pl.ds(sid * _RPT, _RPT)])
        plsc.subcore_barrier()

        def body(i, c):
            off = wid * _EPW + i * _CG
            pltpu.sync_copy(dst_hbm.at[pl.ds(off, _CG)], idx_v)
            pltpu.sync_copy(upd_v, acc_sh.at[idx_v], add=True)
            return c

        lax.fori_loop(0, _NCHUNK, body, 0)
        plsc.subcore_barrier()
        pltpu.sync_copy(acc_sh.at[pl.ds(sid * _RPT, _RPT)],
                        out_hbm.at[pl.ds(cid * _NP + sid * _RPT, _RPT)])

    return k(dst, pattern, zero_n16)


def kernel(x, edge_index, edge_attr, batch, ne_W1, ne_b1, ne_g1, ne_be1,
           ne_W2, ne_b2, ne_g2, ne_be2, conv_W, conv_b, conv_mg, conv_mb,
           conv_ng, conv_nb, outer_g, outer_b, mlp_W1, mlp_b1, mlp_g1,
           mlp_be1, mlp_W2, mlp_b2, mlp_g2, mlp_be2, mlp_W3, mlp_b3):
    src = edge_index[0]
    dst = edge_index[1]
    batch2 = batch.reshape(_N // _BH, 1, _BH)
    row = lambda v: v.reshape(1, -1)
    zero_half = jnp.zeros((_NP, _D // 2), _F32)
    zero_n16 = jnp.zeros((_NP, 16), _F32)
    pattern = jnp.concatenate(
        [jnp.ones((_CG, 1), _F32), jnp.zeros((_CG, 15), _F32)], axis=1)

    h, p = _enc_call(x, ne_W1, row(ne_b1), row(ne_g1), row(ne_be1), ne_W2,
                     row(ne_b2), row(ne_g2), row(ne_be2), conv_W[0, :_D],
                     row(conv_b[0]))
    cntp = _sc_count(dst, pattern, zero_n16).reshape(2, _NP, 16)[:, :_N]

    for i in range(_L):
        g = _sc_gather(p, src)
        m0, m1 = _edge_call(g, edge_attr, conv_W[i, _D:], row(conv_mg[i]),
                            row(conv_mb[i]))
        accp = _sc_scatter(m0, m1, dst, zero_half)
        accl = accp[:_N]
        accr = accp[_NP:_NP + _N]
        j = (i + 1) % _L
        h, p = _comb_call(accl, accr, cntp, h, row(conv_ng[i]),
                          row(conv_nb[i]), row(outer_g[i]), row(outer_b[i]),
                          conv_W[j, :_D], row(conv_b[j]))

    return _head_call(h, batch2, mlp_W1, row(mlp_b1), row(mlp_g1),
                      row(mlp_be1), mlp_W2, row(mlp_b2), row(mlp_g2),
                      row(mlp_be2), mlp_W3, row(mlp_b3))


# speedup vs baseline: 4.5278x; 1.4432x over previous
import functools

import jax
import jax.numpy as jnp
from jax import lax
from jax.experimental import pallas as pl
from jax.experimental.pallas import tpu as pltpu
from jax.experimental.pallas import tpu_sc as plsc

_N = 10000
_E = 320000
_D = 128
_L = 3
_G = 16

_NC = 2
_NS = 16
_NW = _NC * _NS
_EPW = _E // _NW
_CG = 80
_GSTR = 5
_GB = _GSTR * _CG
_SSTR = 2
_SB = _SSTR * _CG
_NP = 10240
_RPT = _NP // _NS

_BN = 2000
_BE = 3200
_BH = 1000

_F32 = jnp.float32
_HIGH = lax.Precision.HIGHEST


def _ln(h, g, b):
    m = jnp.mean(h, axis=-1, keepdims=True)
    v = jnp.mean((h - m) ** 2, axis=-1, keepdims=True)
    return (h - m) * lax.rsqrt(v + 1e-5) * g + b


def _gelu(x):
    return 0.5 * x * (1.0 + lax.erf(x * 0.7071067811865476))


def _enc_body(x_ref, w1_ref, b1_ref, g1_ref, be1_ref, w2_ref, b2_ref, g2_ref,
              be2_ref, wx_ref, bx_ref, h_ref, p_ref):
    t = jnp.dot(x_ref[...], w1_ref[...], precision=_HIGH) + b1_ref[...]
    t = _gelu(_ln(t, g1_ref[...], be1_ref[...]))
    t = jnp.dot(t, w2_ref[...], precision=_HIGH) + b2_ref[...]
    h = _gelu(_ln(t, g2_ref[...], be2_ref[...]))
    h_ref[...] = h
    p_ref[...] = jnp.dot(h, wx_ref[...], precision=_HIGH) + bx_ref[...]


def _edge_body(g_ref, ea_ref, we_ref, mg_ref, mb_ref, m0_ref, m1_ref):
    e2 = jnp.dot(ea_ref[...], we_ref[...], precision=_HIGH)
    m = g_ref[...] + e2
    y = _gelu(_ln(m, mg_ref[...], mb_ref[...]))
    m0_ref[...] = y[:, : _D // 2]
    m1_ref[...] = y[:, _D // 2 :]


def _comb_body(accl_ref, accr_ref, cnt_ref, h_ref, ng_ref, nb_ref, og_ref,
               ob_ref, wx_ref, bx_ref, h2_ref, p_ref):
    s = jnp.concatenate([accl_ref[...], accr_ref[...]], axis=-1)
    cnt = cnt_ref[0, :, 0:1] + cnt_ref[1, :, 0:1]
    agg = s / jnp.maximum(cnt, 1.0)
    h = h_ref[...]
    t = _ln(agg + h, ng_ref[...], nb_ref[...])
    hn = h + _ln(t, og_ref[...], ob_ref[...])
    h2_ref[...] = hn
    p_ref[...] = jnp.dot(hn, wx_ref[...], precision=_HIGH) + bx_ref[...]


def _head_body(h_ref, b_ref, w1_ref, b1_ref, g1_ref, be1_ref, w2_ref, b2_ref,
               g2_ref, be2_ref, w3_ref, b3_ref, out_ref, pool_ref, gc_ref):
    i = pl.program_id(0)
    oh = (lax.broadcasted_iota(jnp.int32, (_G, _BH), 0) == b_ref[0]).astype(_F32)
    part = lax.dot_general(oh, h_ref[...], (((1,), (0,)), ((), ())),
                           precision=_HIGH)
    pc = jnp.sum(oh, axis=1, keepdims=True)

    @pl.when(i == 0)
    def _():
        pool_ref[...] = jnp.zeros_like(pool_ref)
        gc_ref[...] = jnp.zeros_like(gc_ref)

    pool_ref[...] += part
    gc_ref[...] += pc

    @pl.when(i == pl.num_programs(0) - 1)
    def _():
        pooled = pool_ref[...] / jnp.maximum(gc_ref[...], 1.0)
        z = jnp.dot(pooled, w1_ref[...], precision=_HIGH) + b1_ref[...]
        z = jnp.maximum(_ln(z, g1_ref[...], be1_ref[...]), 0.0)
        z = jnp.dot(z, w2_ref[...], precision=_HIGH) + b2_ref[...]
        z = jnp.maximum(_ln(z, g2_ref[...], be2_ref[...]), 0.0)
        out_ref[...] = jnp.dot(z, w3_ref[...], precision=_HIGH) + b3_ref[...]


def _full(shape):
    return pl.BlockSpec(shape, lambda i: (0,) * len(shape))


def _rows(block, width):
    return pl.BlockSpec((block, width), lambda i: (i, 0))


def _enc_call(x, w1, b1, g1, be1, w2, b2, g2, be2, wx, bx):
    full_w = _full((_D, _D))
    full_v = _full((1, _D))
    return pl.pallas_call(
        _enc_body,
        grid=(_N // _BN,),
        in_specs=[_rows(_BN, _D), full_w, full_v, full_v, full_v, full_w,
                  full_v, full_v, full_v, full_w, full_v],
        out_specs=[_rows(_BN, _D), _rows(_BN, _D)],
        out_shape=[jax.ShapeDtypeStruct((_N, _D), _F32),
                   jax.ShapeDtypeStruct((_N, _D), _F32)],
    )(x, w1, b1, g1, be1, w2, b2, g2, be2, wx, bx)


def _edge_call(g, ea, we, mg, mb):
    return pl.pallas_call(
        _edge_body,
        grid=(_E // _BE,),
        in_specs=[_rows(_BE, _D), _rows(_BE, 2), _full((2, _D)),
                  _full((1, _D)), _full((1, _D))],
        out_specs=[_rows(_BE, _D // 2), _rows(_BE, _D // 2)],
        out_shape=[jax.ShapeDtypeStruct((_E, _D // 2), _F32),
                   jax.ShapeDtypeStruct((_E, _D // 2), _F32)],
    )(g, ea, we, mg, mb)


def _comb_call(accl, accr, cnt, h, ng, nb, og, ob, wx, bx):
    full_v = _full((1, _D))
    return pl.pallas_call(
        _comb_body,
        grid=(_N // _BN,),
        in_specs=[_rows(_BN, _D // 2), _rows(_BN, _D // 2),
                  pl.BlockSpec((2, _BN, 16), lambda i: (0, i, 0)),
                  _rows(_BN, _D), full_v, full_v, full_v, full_v,
                  _full((_D, _D)), full_v],
        out_specs=[_rows(_BN, _D), _rows(_BN, _D)],
        out_shape=[jax.ShapeDtypeStruct((_N, _D), _F32),
                   jax.ShapeDtypeStruct((_N, _D), _F32)],
    )(accl, accr, cnt, h, ng, nb, og, ob, wx, bx)


def _head_call(h, batch2, w1, b1, g1, be1, w2, b2, g2, be2, w3, b3):
    return pl.pallas_call(
        _head_body,
        grid=(_N // _BH,),
        in_specs=[_rows(_BH, _D), pl.BlockSpec((1, 1, _BH), lambda i: (i, 0, 0)),
                  _full((_D, _D)), _full((1, _D)), _full((1, _D)),
                  _full((1, _D)), _full((_D, _D // 2)), _full((1, _D // 2)),
                  _full((1, _D // 2)), _full((1, _D // 2)),
                  _full((_D // 2, 1)), _full((1, 1))],
        out_specs=_full((_G, 1)),
        out_shape=jax.ShapeDtypeStruct((_G, 1), _F32),
        scratch_shapes=[pltpu.VMEM((_G, _D), _F32), pltpu.VMEM((_G, 1), _F32)],
    )(h, batch2, w1, b1, g1, be1, w2, b2, g2, be2, w3, b3)


@functools.cache
def _mesh():
    return plsc.VectorSubcoreMesh(core_axis_name="c", subcore_axis_name="s")


def _sc_gather(p, src4):
    ngrp = _EPW // _GB

    @functools.partial(
        pl.kernel, mesh=_mesh(),
        out_type=jax.ShapeDtypeStruct((_E, _D), _F32),
        scratch_types=[pltpu.VMEM((2, _GSTR, _CG), jnp.int32),
                       pltpu.VMEM((_GB, _D), _F32),
                       pltpu.SemaphoreType.DMA,
                       pltpu.SemaphoreType.DMA],
    )
    def k(p_hbm, src_hbm, out_hbm, idx_v, rows_v, isem, gsem):
        wid = lax.axis_index("s") * _NC + lax.axis_index("c")
        base = wid * _EPW

        def idx_start(g, b):
            pltpu.async_copy(src_hbm.at[wid, g], idx_v.at[b], isem)

        def idx_wait(b):
            pltpu.make_async_copy(src_hbm.at[wid, 0], idx_v.at[b], isem).wait()

        def run_group(g, b):
            for j in range(_GSTR):
                pltpu.async_copy(p_hbm.at[idx_v.at[b, j]],
                                 rows_v.at[pl.ds(j * _CG, _CG)], gsem)
            for j in range(_GSTR):
                pltpu.make_async_copy(p_hbm.at[idx_v.at[b, 0]],
                                      rows_v.at[pl.ds(0, _CG)], gsem).wait()
            pltpu.sync_copy(rows_v, out_hbm.at[pl.ds(base + g * _GB, _GB)])

        idx_start(0, 0)

        def body2(k2, c):
            g0 = 2 * k2
            idx_wait(0)
            idx_start(g0 + 1, 1)
            run_group(g0, 0)
            idx_wait(1)
            idx_start(g0 + 2, 0)
            run_group(g0 + 1, 1)
            return c

        lax.fori_loop(0, (ngrp - 1) // 2, body2, 0)
        idx_wait(0)
        run_group(ngrp - 1, 0)

    return k(p, src4)


def _sc_scatter(m0, m1, dst4, zero_half):
    ept = _E // _NS
    ngrp = ept // _SB

    @functools.partial(
        pl.kernel, mesh=_mesh(),
        out_type=jax.ShapeDtypeStruct((2 * _NP, _D // 2), _F32),
        scratch_types=[pltpu.VMEM((2, _SSTR, _CG), jnp.int32),
                       pltpu.VMEM((2, _SB, _D // 2), _F32),
                       pltpu.VMEM_SHARED((_NP, _D // 2), _F32),
                       pltpu.SemaphoreType.DMA,
                       pltpu.SemaphoreType.DMA],
    )
    def k(m0_hbm, m1_hbm, dst_hbm, z_hbm, out_hbm, idx_v, upd_v, acc_sh,
          lsem, ssem):
        cid = lax.axis_index("c")
        sid = lax.axis_index("s")
        pltpu.sync_copy(z_hbm.at[pl.ds(sid * _RPT, _RPT)],
                        acc_sh.at[pl.ds(sid * _RPT, _RPT)])
        plsc.subcore_barrier()

        def half(m_hbm):
            def load_start(g, b):
                pltpu.async_copy(dst_hbm.at[sid, g], idx_v.at[b], lsem)
                pltpu.async_copy(m_hbm.at[pl.ds(sid * ept + g * _SB, _SB)],
                                 upd_v.at[b], lsem)

            def load_wait(b):
                pltpu.make_async_copy(dst_hbm.at[sid, 0], idx_v.at[b],
                                      lsem).wait()
                pltpu.make_async_copy(m_hbm.at[pl.ds(sid * ept, _SB)],
                                      upd_v.at[b], lsem).wait()

            def run_group(b):
                for j in range(_SSTR):
                    pltpu.async_copy(upd_v.at[b, pl.ds(j * _CG, _CG)],
                                     acc_sh.at[idx_v.at[b, j]], ssem,
                                     add=True)
                for j in range(_SSTR):
                    pltpu.make_async_copy(upd_v.at[b, pl.ds(0, _CG)],
                                          acc_sh.at[idx_v.at[b, 0]],
                                          ssem).wait()

            load_start(0, 0)
            load_start(1, 1)
            nit = (ngrp - 1) // 2

            def body2(k2, c):
                g0 = 2 * k2
                load_wait(0)
                run_group(0)
                load_start(g0 + 2, 0)
                load_wait(1)
                run_group(1)

                @pl.when(k2 < nit - 1)
                def _():
                    load_start(g0 + 3, 1)

                return c

            lax.fori_loop(0, nit, body2, 0)
            load_wait(0)
            run_group(0)

        @pl.when(cid == 0)
        def _():
            half(m0_hbm)

        @pl.when(cid == 1)
        def _():
            half(m1_hbm)

        plsc.subcore_barrier()
        pltpu.sync_copy(acc_sh.at[pl.ds(sid * _RPT, _RPT)],
                        out_hbm.at[pl.ds(cid * _NP + sid * _RPT, _RPT)])

    return k(m0, m1, dst4, zero_half)


def _sc_count(dst, pattern, zero_n16):
    @functools.partial(
        pl.kernel, mesh=_mesh(),
        out_type=jax.ShapeDtypeStruct((2 * _NP, 16), _F32),
        scratch_types=[pltpu.VMEM((_CG,), jnp.int32),
                       pltpu.VMEM((_CG, 16), _F32),
                       pltpu.VMEM_SHARED((_NP, 16), _F32)],
    )
    def k(dst_hbm, pat_hbm, z_hbm, out_hbm, idx_v, upd_v, acc_sh):
        cid = lax.axis_index("c")
        sid = lax.axis_index("s")
        wid = sid * _NC + cid
        pltpu.sync_copy(pat_hbm, upd_v)
        pltpu.sync_copy(z_hbm.at[pl.ds(sid * _RPT, _RPT)],
                        acc_sh.at[pl.ds(sid * _RPT, _RPT)])
        plsc.subcore_barrier()

        def body(i, c):
            off = wid * _EPW + i * _CG
            pltpu.sync_copy(dst_hbm.at[pl.ds(off, _CG)], idx_v)
            pltpu.sync_copy(upd_v, acc_sh.at[idx_v], add=True)
            return c

        lax.fori_loop(0, _EPW // _CG, body, 0)
        plsc.subcore_barrier()
        pltpu.sync_copy(acc_sh.at[pl.ds(sid * _RPT, _RPT)],
                        out_hbm.at[pl.ds(cid * _NP + sid * _RPT, _RPT)])

    return k(dst, pattern, zero_n16)


def kernel(x, edge_index, edge_attr, batch, ne_W1, ne_b1, ne_g1, ne_be1,
           ne_W2, ne_b2, ne_g2, ne_be2, conv_W, conv_b, conv_mg, conv_mb,
           conv_ng, conv_nb, outer_g, outer_b, mlp_W1, mlp_b1, mlp_g1,
           mlp_be1, mlp_W2, mlp_b2, mlp_g2, mlp_be2, mlp_W3, mlp_b3):
    src = edge_index[0]
    dst = edge_index[1]
    src4 = src.reshape(_NW, _EPW // _GB, _GSTR, _CG)
    dst4 = dst.reshape(_NS, (_E // _NS) // _SB, _SSTR, _CG)
    batch2 = batch.reshape(_N // _BH, 1, _BH)
    row = lambda v: v.reshape(1, -1)
    zero_half = jnp.zeros((_NP, _D // 2), _F32)
    zero_n16 = jnp.zeros((_NP, 16), _F32)
    pattern = jnp.concatenate(
        [jnp.ones((_CG, 1), _F32), jnp.zeros((_CG, 15), _F32)], axis=1)

    h, p = _enc_call(x, ne_W1, row(ne_b1), row(ne_g1), row(ne_be1), ne_W2,
                     row(ne_b2), row(ne_g2), row(ne_be2), conv_W[0, :_D],
                     row(conv_b[0]))
    cntp = _sc_count(dst, pattern, zero_n16).reshape(2, _NP, 16)[:, :_N]

    for i in range(_L):
        g = _sc_gather(p, src4)
        m0, m1 = _edge_call(g, edge_attr, conv_W[i, _D:], row(conv_mg[i]),
                            row(conv_mb[i]))
        accp = _sc_scatter(m0, m1, dst4, zero_half)
        accl = accp[:_N]
        accr = accp[_NP:_NP + _N]
        j = (i + 1) % _L
        h, p = _comb_call(accl, accr, cntp, h, row(conv_ng[i]),
                          row(conv_nb[i]), row(outer_g[i]), row(outer_b[i]),
                          conv_W[j, :_D], row(conv_b[j]))

    return _head_call(h, batch2, mlp_W1, row(mlp_b1), row(mlp_g1),
                      row(mlp_be1), mlp_W2, row(mlp_b2), row(mlp_g2),
                      row(mlp_be2), mlp_W3, row(mlp_b3))


# speedup vs baseline: 4.5442x; 1.0036x over previous
import functools

import jax
import jax.numpy as jnp
from jax import lax
from jax.experimental import pallas as pl
from jax.experimental.pallas import tpu as pltpu
from jax.experimental.pallas import tpu_sc as plsc

_N = 10000
_E = 320000
_D = 128
_L = 3
_G = 16

_NC = 2
_NS = 16
_NW = _NC * _NS
_EPW = _E // _NW
_CG = 80
_GSTR = 5
_GB = _GSTR * _CG
_SSTR = 2
_SB = _SSTR * _CG
_NP = 10240
_RPT = _NP // _NS

_BN = 2000
_BE = 3200
_BH = 1000

_F32 = jnp.float32
_HIGH = lax.Precision.HIGHEST


def _ln(h, g, b):
    m = jnp.mean(h, axis=-1, keepdims=True)
    v = jnp.mean((h - m) ** 2, axis=-1, keepdims=True)
    return (h - m) * lax.rsqrt(v + 1e-5) * g + b


def _gelu(x):
    return 0.5 * x * (1.0 + lax.erf(x * 0.7071067811865476))


def _enc_body(x_ref, w1_ref, b1_ref, g1_ref, be1_ref, w2_ref, b2_ref, g2_ref,
              be2_ref, wx_ref, bx_ref, h_ref, p_ref):
    t = jnp.dot(x_ref[...], w1_ref[...], precision=_HIGH) + b1_ref[...]
    t = _gelu(_ln(t, g1_ref[...], be1_ref[...]))
    t = jnp.dot(t, w2_ref[...], precision=_HIGH) + b2_ref[...]
    h = _gelu(_ln(t, g2_ref[...], be2_ref[...]))
    h_ref[...] = h
    p_ref[...] = jnp.dot(h, wx_ref[...], precision=_HIGH) + bx_ref[...]


def _edge_body(g_ref, ea_ref, we_ref, mg_ref, mb_ref, m0_ref, m1_ref):
    e2 = jnp.dot(ea_ref[...], we_ref[...], precision=_HIGH)
    m = g_ref[...] + e2
    y = _gelu(_ln(m, mg_ref[...], mb_ref[...]))
    m0_ref[...] = y[:, : _D // 2]
    m1_ref[...] = y[:, _D // 2 :]


def _comb_body(accl_ref, accr_ref, cnt_ref, h_ref, ng_ref, nb_ref, og_ref,
               ob_ref, wx_ref, bx_ref, h2_ref, p_ref):
    s = jnp.concatenate([accl_ref[...], accr_ref[...]], axis=-1)
    cnt = cnt_ref[0, :, 0:1] + cnt_ref[1, :, 0:1]
    agg = s / jnp.maximum(cnt, 1.0)
    h = h_ref[...]
    t = _ln(agg + h, ng_ref[...], nb_ref[...])
    hn = h + _ln(t, og_ref[...], ob_ref[...])
    h2_ref[...] = hn
    p_ref[...] = jnp.dot(hn, wx_ref[...], precision=_HIGH) + bx_ref[...]


def _head_body(h_ref, b_ref, w1_ref, b1_ref, g1_ref, be1_ref, w2_ref, b2_ref,
               g2_ref, be2_ref, w3_ref, b3_ref, out_ref, pool_ref, gc_ref):
    i = pl.program_id(0)
    oh = (lax.broadcasted_iota(jnp.int32, (_G, _BH), 0) == b_ref[0]).astype(_F32)
    part = lax.dot_general(oh, h_ref[...], (((1,), (0,)), ((), ())),
                           precision=_HIGH)
    pc = jnp.sum(oh, axis=1, keepdims=True)

    @pl.when(i == 0)
    def _():
        pool_ref[...] = jnp.zeros_like(pool_ref)
        gc_ref[...] = jnp.zeros_like(gc_ref)

    pool_ref[...] += part
    gc_ref[...] += pc

    @pl.when(i == pl.num_programs(0) - 1)
    def _():
        pooled = pool_ref[...] / jnp.maximum(gc_ref[...], 1.0)
        z = jnp.dot(pooled, w1_ref[...], precision=_HIGH) + b1_ref[...]
        z = jnp.maximum(_ln(z, g1_ref[...], be1_ref[...]), 0.0)
        z = jnp.dot(z, w2_ref[...], precision=_HIGH) + b2_ref[...]
        z = jnp.maximum(_ln(z, g2_ref[...], be2_ref[...]), 0.0)
        out_ref[...] = jnp.dot(z, w3_ref[...], precision=_HIGH) + b3_ref[...]


def _full(shape):
    return pl.BlockSpec(shape, lambda i: (0,) * len(shape))


def _rows(block, width):
    return pl.BlockSpec((block, width), lambda i: (i, 0))


def _enc_call(x, w1, b1, g1, be1, w2, b2, g2, be2, wx, bx):
    full_w = _full((_D, _D))
    full_v = _full((1, _D))
    return pl.pallas_call(
        _enc_body,
        grid=(_N // _BN,),
        in_specs=[_rows(_BN, _D), full_w, full_v, full_v, full_v, full_w,
                  full_v, full_v, full_v, full_w, full_v],
        out_specs=[_rows(_BN, _D), _rows(_BN, _D)],
        out_shape=[jax.ShapeDtypeStruct((_N, _D), _F32),
                   jax.ShapeDtypeStruct((_N, _D), _F32)],
    )(x, w1, b1, g1, be1, w2, b2, g2, be2, wx, bx)


def _edge_call(g, ea, we, mg, mb):
    return pl.pallas_call(
        _edge_body,
        grid=(_E // _BE,),
        in_specs=[_rows(_BE, _D), _rows(_BE, 2), _full((2, _D)),
                  _full((1, _D)), _full((1, _D))],
        out_specs=[_rows(_BE, _D // 2), _rows(_BE, _D // 2)],
        out_shape=[jax.ShapeDtypeStruct((_E, _D // 2), _F32),
                   jax.ShapeDtypeStruct((_E, _D // 2), _F32)],
    )(g, ea, we, mg, mb)


def _comb_call(accl, accr, cnt, h, ng, nb, og, ob, wx, bx):
    full_v = _full((1, _D))
    return pl.pallas_call(
        _comb_body,
        grid=(_N // _BN,),
        in_specs=[_rows(_BN, _D // 2), _rows(_BN, _D // 2),
                  pl.BlockSpec((2, _BN, 16), lambda i: (0, i, 0)),
                  _rows(_BN, _D), full_v, full_v, full_v, full_v,
                  _full((_D, _D)), full_v],
        out_specs=[_rows(_BN, _D), _rows(_BN, _D)],
        out_shape=[jax.ShapeDtypeStruct((_N, _D), _F32),
                   jax.ShapeDtypeStruct((_N, _D), _F32)],
    )(accl, accr, cnt, h, ng, nb, og, ob, wx, bx)


def _head_call(h, batch2, w1, b1, g1, be1, w2, b2, g2, be2, w3, b3):
    return pl.pallas_call(
        _head_body,
        grid=(_N // _BH,),
        in_specs=[_rows(_BH, _D), pl.BlockSpec((1, 1, _BH), lambda i: (i, 0, 0)),
                  _full((_D, _D)), _full((1, _D)), _full((1, _D)),
                  _full((1, _D)), _full((_D, _D // 2)), _full((1, _D // 2)),
                  _full((1, _D // 2)), _full((1, _D // 2)),
                  _full((_D // 2, 1)), _full((1, 1))],
        out_specs=_full((_G, 1)),
        out_shape=jax.ShapeDtypeStruct((_G, 1), _F32),
        scratch_shapes=[pltpu.VMEM((_G, _D), _F32), pltpu.VMEM((_G, 1), _F32)],
    )(h, batch2, w1, b1, g1, be1, w2, b2, g2, be2, w3, b3)


@functools.cache
def _mesh():
    return plsc.VectorSubcoreMesh(core_axis_name="c", subcore_axis_name="s")


def _sc_gather(p, src4):
    ngrp = _EPW // _GB
    _A = 3 * _CG
    _B = 2 * _CG

    @functools.partial(
        pl.kernel, mesh=_mesh(),
        out_type=jax.ShapeDtypeStruct((_E, _D), _F32),
        scratch_types=[pltpu.VMEM((2, _GSTR, _CG), jnp.int32),
                       pltpu.VMEM((_A, _D), _F32),
                       pltpu.VMEM((_B, _D), _F32),
                       pltpu.SemaphoreType.DMA,
                       pltpu.SemaphoreType.DMA,
                       pltpu.SemaphoreType.DMA,
                       pltpu.SemaphoreType.DMA],
    )
    def k(p_hbm, src_hbm, out_hbm, idx_v, rowsa_v, rowsb_v, isem, gsem,
          asem, bsem):
        wid = lax.axis_index("s") * _NC + lax.axis_index("c")
        base = wid * _EPW

        def idx_start(g, b):
            pltpu.async_copy(src_hbm.at[wid, g], idx_v.at[b], isem)

        def idx_wait(b):
            pltpu.make_async_copy(src_hbm.at[wid, 0], idx_v.at[b], isem).wait()

        def wb_a_wait():
            pltpu.make_async_copy(rowsa_v, out_hbm.at[pl.ds(base, _A)],
                                  asem).wait()

        def wb_b_wait():
            pltpu.make_async_copy(rowsb_v, out_hbm.at[pl.ds(base, _B)],
                                  bsem).wait()

        def run_group(g, b, first):
            for j in range(3):
                pltpu.async_copy(p_hbm.at[idx_v.at[b, j]],
                                 rowsa_v.at[pl.ds(j * _CG, _CG)], gsem)
            for j in range(3):
                pltpu.make_async_copy(p_hbm.at[idx_v.at[b, 0]],
                                      rowsa_v.at[pl.ds(0, _CG)], gsem).wait()
            if not first:
                wb_b_wait()
            pltpu.async_copy(rowsa_v, out_hbm.at[pl.ds(base + g * _GB, _A)],
                             asem)
            for j in range(3, _GSTR):
                pltpu.async_copy(p_hbm.at[idx_v.at[b, j]],
                                 rowsb_v.at[pl.ds((j - 3) * _CG, _CG)], gsem)
            for j in range(3, _GSTR):
                pltpu.make_async_copy(p_hbm.at[idx_v.at[b, 0]],
                                      rowsb_v.at[pl.ds(0, _CG)], gsem).wait()
            wb_a_wait()
            pltpu.async_copy(rowsb_v,
                             out_hbm.at[pl.ds(base + g * _GB + _A, _B)], bsem)

        idx_start(0, 0)
        idx_wait(0)
        idx_start(1, 1)
        run_group(0, 0, True)
        idx_wait(1)
        idx_start(2, 0)
        run_group(1, 1, False)

        def body2(k2, c):
            g0 = 2 * k2
            idx_wait(0)
            idx_start(g0 + 1, 1)
            run_group(g0, 0, False)
            idx_wait(1)
            idx_start(g0 + 2, 0)
            run_group(g0 + 1, 1, False)
            return c

        lax.fori_loop(1, (ngrp - 1) // 2, body2, 0)
        idx_wait(0)
        run_group(ngrp - 1, 0, False)
        wb_b_wait()

    return k(p, src4)


def _sc_scatter(m0, m1, dst4, zero_half):
    ept = _E // _NS
    ngrp = ept // _SB

    @functools.partial(
        pl.kernel, mesh=_mesh(),
        out_type=jax.ShapeDtypeStruct((2 * _NP, _D // 2), _F32),
        scratch_types=[pltpu.VMEM((2, _SSTR, _CG), jnp.int32),
                       pltpu.VMEM((2, _SB, _D // 2), _F32),
                       pltpu.VMEM_SHARED((_NP, _D // 2), _F32),
                       pltpu.SemaphoreType.DMA,
                       pltpu.SemaphoreType.DMA],
    )
    def k(m0_hbm, m1_hbm, dst_hbm, z_hbm, out_hbm, idx_v, upd_v, acc_sh,
          lsem, ssem):
        cid = lax.axis_index("c")
        sid = lax.axis_index("s")
        pltpu.sync_copy(z_hbm.at[pl.ds(sid * _RPT, _RPT)],
                        acc_sh.at[pl.ds(sid * _RPT, _RPT)])
        plsc.subcore_barrier()

        def half(m_hbm):
            def load_start(g, b):
                pltpu.async_copy(dst_hbm.at[sid, g], idx_v.at[b], lsem)
                pltpu.async_copy(m_hbm.at[pl.ds(sid * ept + g * _SB, _SB)],
                                 upd_v.at[b], lsem)

            def load_wait(b):
                pltpu.make_async_copy(dst_hbm.at[sid, 0], idx_v.at[b],
                                      lsem).wait()
                pltpu.make_async_copy(m_hbm.at[pl.ds(sid * ept, _SB)],
                                      upd_v.at[b], lsem).wait()

            def run_group(b):
                for j in range(_SSTR):
                    pltpu.async_copy(upd_v.at[b, pl.ds(j * _CG, _CG)],
                                     acc_sh.at[idx_v.at[b, j]], ssem,
                                     add=True)
                for j in range(_SSTR):
                    pltpu.make_async_copy(upd_v.at[b, pl.ds(0, _CG)],
                                          acc_sh.at[idx_v.at[b, 0]],
                                          ssem).wait()

            load_start(0, 0)
            load_start(1, 1)
            nit = (ngrp - 1) // 2

            def body2(k2, c):
                g0 = 2 * k2
                load_wait(0)
                run_group(0)
                load_start(g0 + 2, 0)
                load_wait(1)
                run_group(1)

                @pl.when(k2 < nit - 1)
                def _():
                    load_start(g0 + 3, 1)

                return c

            lax.fori_loop(0, nit, body2, 0)
            load_wait(0)
            run_group(0)

        @pl.when(cid == 0)
        def _():
            half(m0_hbm)

        @pl.when(cid == 1)
        def _():
            half(m1_hbm)

        plsc.subcore_barrier()
        pltpu.sync_copy(acc_sh.at[pl.ds(sid * _RPT, _RPT)],
                        out_hbm.at[pl.ds(cid * _NP + sid * _RPT, _RPT)])

    return k(m0, m1, dst4, zero_half)


def _sc_count(dst, pattern, zero_n16):
    @functools.partial(
        pl.kernel, mesh=_mesh(),
        out_type=jax.ShapeDtypeStruct((2 * _NP, 16), _F32),
        scratch_types=[pltpu.VMEM((_CG,), jnp.int32),
                       pltpu.VMEM((_CG, 16), _F32),
                       pltpu.VMEM_SHARED((_NP, 16), _F32)],
    )
    def k(dst_hbm, pat_hbm, z_hbm, out_hbm, idx_v, upd_v, acc_sh):
        cid = lax.axis_index("c")
        sid = lax.axis_index("s")
        wid = sid * _NC + cid
        pltpu.sync_copy(pat_hbm, upd_v)
        pltpu.sync_copy(z_hbm.at[pl.ds(sid * _RPT, _RPT)],
                        acc_sh.at[pl.ds(sid * _RPT, _RPT)])
        plsc.subcore_barrier()

        def body(i, c):
            off = wid * _EPW + i * _CG
            pltpu.sync_copy(dst_hbm.at[pl.ds(off, _CG)], idx_v)
            pltpu.sync_copy(upd_v, acc_sh.at[idx_v], add=True)
            return c

        lax.fori_loop(0, _EPW // _CG, body, 0)
        plsc.subcore_barrier()
        pltpu.sync_copy(acc_sh.at[pl.ds(sid * _RPT, _RPT)],
                        out_hbm.at[pl.ds(cid * _NP + sid * _RPT, _RPT)])

    return k(dst, pattern, zero_n16)


def kernel(x, edge_index, edge_attr, batch, ne_W1, ne_b1, ne_g1, ne_be1,
           ne_W2, ne_b2, ne_g2, ne_be2, conv_W, conv_b, conv_mg, conv_mb,
           conv_ng, conv_nb, outer_g, outer_b, mlp_W1, mlp_b1, mlp_g1,
           mlp_be1, mlp_W2, mlp_b2, mlp_g2, mlp_be2, mlp_W3, mlp_b3):
    src = edge_index[0]
    dst = edge_index[1]
    src4 = src.reshape(_NW, _EPW // _GB, _GSTR, _CG)
    dst4 = dst.reshape(_NS, (_E // _NS) // _SB, _SSTR, _CG)
    batch2 = batch.reshape(_N // _BH, 1, _BH)
    row = lambda v: v.reshape(1, -1)
    zero_half = jnp.zeros((_NP, _D // 2), _F32)
    zero_n16 = jnp.zeros((_NP, 16), _F32)
    pattern = jnp.concatenate(
        [jnp.ones((_CG, 1), _F32), jnp.zeros((_CG, 15), _F32)], axis=1)

    h, p = _enc_call(x, ne_W1, row(ne_b1), row(ne_g1), row(ne_be1), ne_W2,
                     row(ne_b2), row(ne_g2), row(ne_be2), conv_W[0, :_D],
                     row(conv_b[0]))
    cntp = _sc_count(dst, pattern, zero_n16).reshape(2, _NP, 16)[:, :_N]

    for i in range(_L):
        g = _sc_gather(p, src4)
        m0, m1 = _edge_call(g, edge_attr, conv_W[i, _D:], row(conv_mg[i]),
                            row(conv_mb[i]))
        accp = _sc_scatter(m0, m1, dst4, zero_half)
        accl = accp[:_N]
        accr = accp[_NP:_NP + _N]
        j = (i + 1) % _L
        h, p = _comb_call(accl, accr, cntp, h, row(conv_ng[i]),
                          row(conv_nb[i]), row(outer_g[i]), row(outer_b[i]),
                          conv_W[j, :_D], row(conv_b[j]))

    return _head_call(h, batch2, mlp_W1, row(mlp_b1), row(mlp_g1),
                      row(mlp_be1), mlp_W2, row(mlp_b2), row(mlp_g2),
                      row(mlp_be2), mlp_W3, row(mlp_b3))
